# baseline (speedup 1.0000x reference)
import functools

import jax
import jax.numpy as jnp
from jax import lax
from jax.experimental import pallas as pl
from jax.experimental.pallas import tpu as pltpu
from jax.experimental.pallas import tpu_sc as plsc

N_CLASS = 80
N_ANCHOR = 3
G = 76
GG = G * G
B = 16
CH = N_CLASS + 5
STRIDE = 8.0
THRESH = 0.5
NO_OBJ_W = 100.0
NCELL = float(B * N_ANCHOR * GG)


def _tc_body(x_ref, anch_ref, out_ref, sum_ref):
    b = pl.program_id(0)
    a = pl.program_id(1)
    X = x_ref[0, 0]
    S = jax.nn.sigmoid(X)
    col = lax.broadcasted_iota(jnp.int32, (1, GG), 1)
    gx = (col % G).astype(jnp.float32)
    gy = (col // G).astype(jnp.float32)
    aw = anch_ref[a, 0]
    ah = anch_ref[a, 1]
    r0 = (S[0:1] + gx) * STRIDE
    r1 = (S[1:2] + gy) * STRIDE
    r2 = jnp.exp(S[2:3]) * (aw * STRIDE)
    r3 = jnp.exp(S[3:4]) * (ah * STRIDE)
    top = jnp.concatenate([r0, r1, r2, r3, S[4:]], axis=0)
    out_ref[0, 0] = top.T
    z = X[4:5]
    sp = jnp.maximum(z, 0.0) + jnp.log1p(jnp.exp(-jnp.abs(z)))
    part = jnp.sum(jnp.minimum(sp, 100.0))

    @pl.when((b == 0) & (a == 0))
    def _():
        sum_ref[0, 0] = 0.0

    sum_ref[0, 0] += part


def _tc_transform(x4, anchors, interpret=False):
    return pl.pallas_call(
        _tc_body,
        grid=(B, N_ANCHOR),
        in_specs=[
            pl.BlockSpec((1, 1, CH, GG), lambda b, a: (b, a, 0, 0)),
            pl.BlockSpec(memory_space=pltpu.SMEM),
        ],
        out_specs=[
            pl.BlockSpec((1, 1, GG, CH), lambda b, a: (b, a, 0, 0)),
            pl.BlockSpec(memory_space=pltpu.SMEM),
        ],
        out_shape=[
            jax.ShapeDtypeStruct((B, N_ANCHOR, GG, CH), jnp.float32),
            jax.ShapeDtypeStruct((1, 1), jnp.float32),
        ],
        interpret=interpret,
    )(x4, anchors)


def _poly_log1p(w):
    s = w / (2.0 + w)
    s2 = s * s
    return 2.0 * s * (1.0 + s2 * (1.0 / 3.0 + s2 * (0.2 + s2 * (1.0 / 7.0 + s2 / 9.0))))


def _poly_log(x):
    bits = lax.bitcast_convert_type(x, jnp.int32)
    e = ((bits >> 23) & 0xFF) - 127
    mbits = (bits & 0x7FFFFF) | (127 << 23)
    m = lax.bitcast_convert_type(mbits, jnp.float32)
    big = m > 1.4142135
    m = jnp.where(big, m * 0.5, m)
    e = e + jnp.where(big, 1, 0)
    s = (m - 1.0) / (m + 1.0)
    s2 = s * s
    lm = 2.0 * s * (1.0 + s2 * (1.0 / 3.0 + s2 * (0.2 + s2 * (1.0 / 7.0 + s2 / 9.0))))
    return e.astype(jnp.float32) * 0.6931471805599453 + lm


def _softplus_nc(t):
    w = jnp.exp(-jnp.abs(t))
    return jnp.maximum(t, 0.0) + _poly_log1p(w)


def _softplus_c(t):
    return jnp.minimum(_softplus_nc(t), 100.0)


def _sigmoid_c(z):
    return 1.0 / (1.0 + jnp.exp(-z))


_NROW = CH + N_ANCHOR
_NIDX = _NROW * B


def _vsum(v):
    iot = lax.iota(jnp.int32, 16)
    for sh in (8, 4, 2, 1):
        perm = (iot + sh) & 15
        v = v + lax.gather(
            v, perm[:, None],
            lax.GatherDimensionNumbers(offset_dims=(), collapsed_slice_dims=(0,),
                                       start_index_map=(0,)),
            (1,), mode=lax.GatherScatterMode.PROMISE_IN_BOUNDS)
    return v


def _sc_body(x1, tgt, anc, out, tv, av, idxr, buf, resv, sem):
    wid = lax.axis_index("s") * 2 + lax.axis_index("c")

    @pl.when(wid == 0)
    def _():
        pltpu.sync_copy(tgt, tv)
        pltpu.sync_copy(anc, av)
        iot = lax.iota(jnp.int32, 16)
        f32 = jnp.float32

        tcx = tv[0, :] * float(G)
        tcy = tv[1, :] * float(G)
        tw = tv[2, :]
        th = tv[3, :]
        tci = tcx.astype(jnp.int32)
        tcj = tcy.astype(jnp.int32)
        fx = tcx - tci.astype(f32)
        fy = tcy - tcj.astype(f32)
        colc = tci * G + tcj
        ious = []
        for a in range(N_ANCHOR):
            aw = av[a, :]
            ah = av[N_ANCHOR + a, :]
            inter = jnp.minimum(aw, tw) * jnp.minimum(ah, th)
            union = tw * th + aw * ah - inter
            ious.append(inter / union)
        best = jnp.where(ious[1] > ious[0], 1, 0)
        best = jnp.where(ious[2] > jnp.maximum(ious[0], ious[1]), 2, best)
        aw_b = jnp.where(best == 0, av[0, :], jnp.where(best == 1, av[1, :], av[2, :]))
        ah_b = jnp.where(best == 0, av[3, :], jnp.where(best == 1, av[4, :], av[5, :]))

        fbase = (iot * (N_ANCHOR * CH) + best * CH) * GG + colc
        for j in range(CH):
            idxr[pl.ds(j * 16, 16)] = fbase + j * GG
        for a in range(N_ANCHOR):
            idxr[pl.ds((CH + a) * 16, 16)] = (
                iot * (N_ANCHOR * CH) + (a * CH + 4)) * GG + colc
        copies = []
        for ci in range(_NIDX // 128):
            copies.append(pltpu.async_copy(
                x1.at[idxr.at[pl.ds(ci * 128, 128)]],
                buf.at[pl.ds(ci * 128, 128)], sem))
        for cp in copies:
            cp.wait()

        def val(j):
            return buf[pl.ds(j * 16, 16)]

        d0 = _sigmoid_c(val(0)) - fx
        d1 = _sigmoid_c(val(1)) - fy
        d2 = _sigmoid_c(val(2)) - _poly_log(tw / aw_b + 1e-16)
        d3 = _sigmoid_c(val(3)) - _poly_log(th / ah_b + 1e-16)
        box_sum = _vsum(d0 * d0 + d1 * d1 + d2 * d2 + d3 * d3)
        objconf_sum = _vsum(_softplus_c(-val(4)))

        cacc = jnp.zeros((16,), f32)
        for j in range(5, CH):
            v = val(j)
            t = tv[j - 1, :]
            a_sp = _softplus_nc(-v)
            cacc = cacc + t * jnp.minimum(a_sp, 100.0) + (1.0 - t) * jnp.minimum(v + a_sp, 100.0)
        class_sum = _vsum(cacc)

        corr = jnp.zeros((16,), f32)
        cnt = jnp.zeros((16,), f32)
        for a in range(N_ANCHOR):
            z = val(CH + a)
            rem = (ious[a] > THRESH) | (best == a)
            corr = corr + jnp.where(rem, _softplus_c(z), 0.0)
            cnt = cnt + jnp.where(rem, 1.0, 0.0)
        corr_sum = _vsum(corr)
        removed = _vsum(cnt)

        sel = lambda k: jnp.where(iot == k, 1.0, 0.0)
        resv[...] = (box_sum * sel(0) + objconf_sum * sel(1) + class_sum * sel(2)
                     + corr_sum * sel(3) + removed * sel(4))
        pltpu.sync_copy(resv, out)


def _make_sc_loss():
    return functools.partial(
        pl.kernel,
        out_type=jax.ShapeDtypeStruct((16,), jnp.float32),
        mesh=plsc.VectorSubcoreMesh(core_axis_name="c", subcore_axis_name="s"),
        scratch_types=[
            pltpu.VMEM((84, B), jnp.float32),
            pltpu.VMEM((2 * N_ANCHOR, B), jnp.float32),
            pltpu.VMEM((_NIDX,), jnp.int32),
            pltpu.VMEM((_NIDX,), jnp.float32),
            pltpu.VMEM((16,), jnp.float32),
            pltpu.SemaphoreType.DMA,
        ],
    )(_sc_body)


def _loss_parts_jnp(x, target, anchors):
    t0 = target[:, 0]; t1 = target[:, 1]; tw = target[:, 2]; th = target[:, 3]
    tcx = t0 * G; tcy = t1 * G
    tci = tcx.astype(jnp.int32); tcj = tcy.astype(jnp.int32)
    fx = tcx - tci.astype(jnp.float32); fy = tcy - tcj.astype(jnp.float32)
    colc = tci * G + tcj
    ious = []
    for a in range(N_ANCHOR):
        inter = jnp.minimum(anchors[a, 0], tw) * jnp.minimum(anchors[a, 1], th)
        union = tw * th + anchors[a, 0] * anchors[a, 1] - inter
        ious.append(inter / union)
    best = jnp.where(ious[1] > ious[0], 1, 0)
    best = jnp.where(ious[2] > jnp.maximum(ious[0], ious[1]), 2, best)
    aw_b = jnp.where(best == 0, anchors[0, 0], jnp.where(best == 1, anchors[1, 0], anchors[2, 0]))
    ah_b = jnp.where(best == 0, anchors[0, 1], jnp.where(best == 1, anchors[1, 1], anchors[2, 1]))
    bi = jnp.arange(B)
    xf = x.reshape(B, N_ANCHOR * CH, GG)

    def val(j):
        return xf[bi, best * CH + j, colc]

    d0 = _sigmoid_c(val(0)) - fx
    d1 = _sigmoid_c(val(1)) - fy
    d2 = _sigmoid_c(val(2)) - _poly_log(tw / aw_b + 1e-16)
    d3 = _sigmoid_c(val(3)) - _poly_log(th / ah_b + 1e-16)
    box_sum = jnp.sum(d0 * d0 + d1 * d1 + d2 * d2 + d3 * d3)
    objconf_sum = jnp.sum(_softplus_c(-val(4)))
    acc = jnp.zeros((B,), jnp.float32)
    for j in range(5, CH):
        v = val(j)
        t = target[:, j - 1]
        acc = acc + t * _softplus_c(-v) + (1.0 - t) * _softplus_c(v)
    class_sum = jnp.sum(acc)
    corr = jnp.zeros((B,), jnp.float32)
    cnt = jnp.zeros((B,), jnp.float32)
    for a in range(N_ANCHOR):
        zc = xf[bi, a * CH + 4, colc]
        rem = (ious[a] > THRESH) | (best == a)
        corr = corr + jnp.where(rem, _softplus_c(zc), 0.0)
        cnt = cnt + rem.astype(jnp.float32)
    return box_sum, objconf_sum, class_sum, jnp.sum(corr), jnp.sum(cnt)


def kernel(x, target, anchors):
    x4 = x.reshape(B, N_ANCHOR, CH, GG)
    out4, s_total = _tc_transform(x4, anchors)
    output = out4.reshape(B, N_ANCHOR * GG, CH)
    parts = _make_sc_loss()(
        x.reshape(B * N_ANCHOR * CH * GG), target.T,
        jnp.broadcast_to(anchors.T.reshape(2 * N_ANCHOR, 1), (2 * N_ANCHOR, B)))
    box_sum, objconf_sum, class_sum, corr_sum, removed = (
        parts[0], parts[1], parts[2], parts[3], parts[4])
    loss = (box_sum / B + objconf_sum / B
            + NO_OBJ_W * (s_total[0, 0] - corr_sum) / (NCELL - removed)
            + class_sum / (B * N_CLASS))
    return output, loss


# speedup vs baseline: 4.2471x; 4.2471x over previous
import functools

import jax
import jax.numpy as jnp
from jax import lax
from jax.experimental import pallas as pl
from jax.experimental.pallas import tpu as pltpu
from jax.experimental.pallas import tpu_sc as plsc

N_CLASS = 80
N_ANCHOR = 3
G = 76
GG = G * G
B = 16
CH = N_CLASS + 5
STRIDE = 8.0
THRESH = 0.5
NO_OBJ_W = 100.0
NCELL = float(B * N_ANCHOR * GG)


def _tc_body(xa_ref, xb_ref, xc_ref, tgt_ref, anch_ref, out_ref, sum_ref, side_ref):
    g = pl.program_id(0)
    A0 = xa_ref[...]
    A1 = xb_ref[...]
    A2 = xc_ref[...]
    T = tgt_ref[...]
    tci = jnp.floor(T[:, 0:1] * float(G)).astype(jnp.int32).reshape(B, 1, 1, 1)
    tcj = jnp.floor(T[:, 1:2] * float(G)).astype(jnp.int32).reshape(B, 1, 1, 1)
    g1 = lax.broadcasted_iota(jnp.int32, (1, 1, G, G), 2)
    g2 = lax.broadcasted_iota(jnp.int32, (1, 1, G, G), 3)
    cmask = jnp.where((g1 == tci) & (g2 == tcj), 1.0, 0.0)

    def colsum(A):
        return jnp.sum(jnp.sum(A * cmask, axis=3), axis=2).T

    side_ref[...] = jnp.concatenate(
        [colsum(A0)[..., None], colsum(A1)[..., None], colsum(A2)[..., None]],
        axis=2)

    def flat(P):
        return P.transpose(1, 0, 2, 3).reshape(5, B, GG)

    S0 = jax.nn.sigmoid(A0)
    S1 = jax.nn.sigmoid(A1)
    S2 = jax.nn.sigmoid(A2)

    @pl.when(g > 0)
    def _():
        out_ref[...] = jnp.concatenate([flat(S0), flat(S1), flat(S2)], axis=2)

    @pl.when(g == 0)
    def _():
        gxf = g2.astype(jnp.float32)
        gyf = g1.astype(jnp.float32)

        def head(S, a):
            r0 = (S[:, 0:1] + gxf) * STRIDE
            r1 = (S[:, 1:2] + gyf) * STRIDE
            r2 = jnp.exp(S[:, 2:3]) * (anch_ref[a, 0] * STRIDE)
            r3 = jnp.exp(S[:, 3:4]) * (anch_ref[a, 1] * STRIDE)
            return flat(jnp.concatenate([r0, r1, r2, r3, S[:, 4:5]], axis=1))

        out_ref[...] = jnp.concatenate([head(S0, 0), head(S1, 1), head(S2, 2)],
                                       axis=2)

        def sptot(A):
            z = A[:, 4:5]
            sp = jnp.maximum(z, 0.0) + jnp.log1p(jnp.exp(-jnp.abs(z)))
            return jnp.sum(jnp.minimum(sp, 100.0))

        sum_ref[0, 0] = sptot(A0) + sptot(A1) + sptot(A2)


def _tc_transform(x, anchors, target, interpret=False):
    xv = x.reshape(B, N_ANCHOR * CH, G, G)
    return pl.pallas_call(
        _tc_body,
        grid=(CH // 5,),
        in_specs=[
            pl.BlockSpec((B, 5, G, G), lambda g: (0, g, 0, 0)),
            pl.BlockSpec((B, 5, G, G), lambda g: (0, CH // 5 + g, 0, 0)),
            pl.BlockSpec((B, 5, G, G), lambda g: (0, 2 * (CH // 5) + g, 0, 0)),
            pl.BlockSpec(memory_space=pltpu.VMEM),
            pl.BlockSpec(memory_space=pltpu.SMEM),
        ],
        out_specs=[
            pl.BlockSpec((5, B, N_ANCHOR * GG), lambda g: (g, 0, 0)),
            pl.BlockSpec(memory_space=pltpu.SMEM),
            pl.BlockSpec((5, B, N_ANCHOR), lambda g: (g, 0, 0)),
        ],
        out_shape=[
            jax.ShapeDtypeStruct((CH, B, N_ANCHOR * GG), jnp.float32),
            jax.ShapeDtypeStruct((1, 1), jnp.float32),
            jax.ShapeDtypeStruct((CH, B, N_ANCHOR), jnp.float32),
        ],
        interpret=interpret,
    )(xv, xv, xv, target, anchors)


def _poly_log1p(w):
    s = w / (2.0 + w)
    s2 = s * s
    return 2.0 * s * (1.0 + s2 * (1.0 / 3.0 + s2 * (0.2 + s2 * (1.0 / 7.0 + s2 / 9.0))))


def _poly_log(x):
    bits = lax.bitcast_convert_type(x, jnp.int32)
    e = ((bits >> 23) & 0xFF) - 127
    mbits = (bits & 0x7FFFFF) | (127 << 23)
    m = lax.bitcast_convert_type(mbits, jnp.float32)
    big = m > 1.4142135
    m = jnp.where(big, m * 0.5, m)
    e = e + jnp.where(big, 1, 0)
    s = (m - 1.0) / (m + 1.0)
    s2 = s * s
    lm = 2.0 * s * (1.0 + s2 * (1.0 / 3.0 + s2 * (0.2 + s2 * (1.0 / 7.0 + s2 / 9.0))))
    return e.astype(jnp.float32) * 0.6931471805599453 + lm


def _softplus_nc(t):
    w = jnp.exp(-jnp.abs(t))
    return jnp.maximum(t, 0.0) + _poly_log1p(w)


def _softplus_c(t):
    return jnp.minimum(_softplus_nc(t), 100.0)


def _sigmoid_c(z):
    return 1.0 / (1.0 + jnp.exp(-z))


_NROW = CH + N_ANCHOR
_NIDX = _NROW * B


def _vsum(v):
    iot = lax.iota(jnp.int32, 16)
    for sh in (8, 4, 2, 1):
        perm = (iot + sh) & 15
        v = v + lax.gather(
            v, perm[:, None],
            lax.GatherDimensionNumbers(offset_dims=(), collapsed_slice_dims=(0,),
                                       start_index_map=(0,)),
            (1,), mode=lax.GatherScatterMode.PROMISE_IN_BOUNDS)
    return v


def _sc_body(x1, tgt, anc, out, tv, av, idxr, buf, resv, sem):
    wid = lax.axis_index("s") * 2 + lax.axis_index("c")

    @pl.when(wid == 0)
    def _():
        pltpu.sync_copy(tgt, tv)
        pltpu.sync_copy(anc, av)
        iot = lax.iota(jnp.int32, 16)
        f32 = jnp.float32

        tcx = tv[0, :] * float(G)
        tcy = tv[1, :] * float(G)
        tw = tv[2, :]
        th = tv[3, :]
        tci = tcx.astype(jnp.int32)
        tcj = tcy.astype(jnp.int32)
        fx = tcx - tci.astype(f32)
        fy = tcy - tcj.astype(f32)
        colc = tci * G + tcj
        ious = []
        for a in range(N_ANCHOR):
            aw = av[a, :]
            ah = av[N_ANCHOR + a, :]
            inter = jnp.minimum(aw, tw) * jnp.minimum(ah, th)
            union = tw * th + aw * ah - inter
            ious.append(inter / union)
        best = jnp.where(ious[1] > ious[0], 1, 0)
        best = jnp.where(ious[2] > jnp.maximum(ious[0], ious[1]), 2, best)
        aw_b = jnp.where(best == 0, av[0, :], jnp.where(best == 1, av[1, :], av[2, :]))
        ah_b = jnp.where(best == 0, av[3, :], jnp.where(best == 1, av[4, :], av[5, :]))

        fbase = iot * N_ANCHOR + best
        for j in range(CH):
            idxr[pl.ds(j * 16, 16)] = fbase + j * (B * N_ANCHOR)
        for a in range(N_ANCHOR):
            idxr[pl.ds((CH + a) * 16, 16)] = 4 * (B * N_ANCHOR) + iot * N_ANCHOR + a
        copies = []
        for ci in range(_NIDX // 128):
            copies.append(pltpu.async_copy(
                x1.at[idxr.at[pl.ds(ci * 128, 128)]],
                buf.at[pl.ds(ci * 128, 128)], sem))
        for cp in copies:
            cp.wait()

        def val(j):
            return buf[pl.ds(j * 16, 16)]

        d0 = _sigmoid_c(val(0)) - fx
        d1 = _sigmoid_c(val(1)) - fy
        d2 = _sigmoid_c(val(2)) - _poly_log(tw / aw_b + 1e-16)
        d3 = _sigmoid_c(val(3)) - _poly_log(th / ah_b + 1e-16)
        box_sum = _vsum(d0 * d0 + d1 * d1 + d2 * d2 + d3 * d3)
        objconf_sum = _vsum(_softplus_c(-val(4)))

        cacc = jnp.zeros((16,), f32)
        for j in range(5, CH):
            v = val(j)
            t = tv[j - 1, :]
            a_sp = _softplus_nc(-v)
            cacc = cacc + t * jnp.minimum(a_sp, 100.0) + (1.0 - t) * jnp.minimum(v + a_sp, 100.0)
        class_sum = _vsum(cacc)

        corr = jnp.zeros((16,), f32)
        cnt = jnp.zeros((16,), f32)
        for a in range(N_ANCHOR):
            z = val(CH + a)
            rem = (ious[a] > THRESH) | (best == a)
            corr = corr + jnp.where(rem, _softplus_c(z), 0.0)
            cnt = cnt + jnp.where(rem, 1.0, 0.0)
        corr_sum = _vsum(corr)
        removed = _vsum(cnt)

        sel = lambda k: jnp.where(iot == k, 1.0, 0.0)
        resv[...] = (box_sum * sel(0) + objconf_sum * sel(1) + class_sum * sel(2)
                     + corr_sum * sel(3) + removed * sel(4))
        pltpu.sync_copy(resv, out)


def _make_sc_loss():
    return functools.partial(
        pl.kernel,
        out_type=jax.ShapeDtypeStruct((16,), jnp.float32),
        mesh=plsc.VectorSubcoreMesh(core_axis_name="c", subcore_axis_name="s"),
        scratch_types=[
            pltpu.VMEM((84, B), jnp.float32),
            pltpu.VMEM((2 * N_ANCHOR, B), jnp.float32),
            pltpu.VMEM((_NIDX,), jnp.int32),
            pltpu.VMEM((_NIDX,), jnp.float32),
            pltpu.VMEM((16,), jnp.float32),
            pltpu.SemaphoreType.DMA,
        ],
    )(_sc_body)


def _loss_parts_jnp(x, target, anchors):
    t0 = target[:, 0]; t1 = target[:, 1]; tw = target[:, 2]; th = target[:, 3]
    tcx = t0 * G; tcy = t1 * G
    tci = tcx.astype(jnp.int32); tcj = tcy.astype(jnp.int32)
    fx = tcx - tci.astype(jnp.float32); fy = tcy - tcj.astype(jnp.float32)
    colc = tci * G + tcj
    ious = []
    for a in range(N_ANCHOR):
        inter = jnp.minimum(anchors[a, 0], tw) * jnp.minimum(anchors[a, 1], th)
        union = tw * th + anchors[a, 0] * anchors[a, 1] - inter
        ious.append(inter / union)
    best = jnp.where(ious[1] > ious[0], 1, 0)
    best = jnp.where(ious[2] > jnp.maximum(ious[0], ious[1]), 2, best)
    aw_b = jnp.where(best == 0, anchors[0, 0], jnp.where(best == 1, anchors[1, 0], anchors[2, 0]))
    ah_b = jnp.where(best == 0, anchors[0, 1], jnp.where(best == 1, anchors[1, 1], anchors[2, 1]))
    bi = jnp.arange(B)
    xf = x.reshape(B, N_ANCHOR * CH, GG)

    def val(j):
        return xf[bi, best * CH + j, colc]

    d0 = _sigmoid_c(val(0)) - fx
    d1 = _sigmoid_c(val(1)) - fy
    d2 = _sigmoid_c(val(2)) - _poly_log(tw / aw_b + 1e-16)
    d3 = _sigmoid_c(val(3)) - _poly_log(th / ah_b + 1e-16)
    box_sum = jnp.sum(d0 * d0 + d1 * d1 + d2 * d2 + d3 * d3)
    objconf_sum = jnp.sum(_softplus_c(-val(4)))
    acc = jnp.zeros((B,), jnp.float32)
    for j in range(5, CH):
        v = val(j)
        t = target[:, j - 1]
        acc = acc + t * _softplus_c(-v) + (1.0 - t) * _softplus_c(v)
    class_sum = jnp.sum(acc)
    corr = jnp.zeros((B,), jnp.float32)
    cnt = jnp.zeros((B,), jnp.float32)
    for a in range(N_ANCHOR):
        zc = xf[bi, a * CH + 4, colc]
        rem = (ious[a] > THRESH) | (best == a)
        corr = corr + jnp.where(rem, _softplus_c(zc), 0.0)
        cnt = cnt + rem.astype(jnp.float32)
    return box_sum, objconf_sum, class_sum, jnp.sum(corr), jnp.sum(cnt)


def kernel(x, target, anchors):
    out_r, s_total, side = _tc_transform(x, anchors, target)
    output = jnp.transpose(out_r, (1, 2, 0))
    parts = _make_sc_loss()(
        side.reshape(CH * B * N_ANCHOR), target.T,
        jnp.broadcast_to(anchors.T.reshape(2 * N_ANCHOR, 1), (2 * N_ANCHOR, B)))
    box_sum, objconf_sum, class_sum, corr_sum, removed = (
        parts[0], parts[1], parts[2], parts[3], parts[4])
    loss = (box_sum / B + objconf_sum / B
            + NO_OBJ_W * (s_total[0, 0] - corr_sum) / (NCELL - removed)
            + class_sum / (B * N_CLASS))
    return output, loss


# speedup vs baseline: 4.2472x; 1.0000x over previous
import functools

import jax
import jax.numpy as jnp
from jax import lax
from jax.experimental import pallas as pl
from jax.experimental.pallas import tpu as pltpu
from jax.experimental.pallas import tpu_sc as plsc

N_CLASS = 80
N_ANCHOR = 3
G = 76
GG = G * G
B = 16
CH = N_CLASS + 5
STRIDE = 8.0
THRESH = 0.5
NO_OBJ_W = 100.0
NCELL = float(B * N_ANCHOR * GG)


def _tc_body(xa_ref, xb_ref, xc_ref, tgt_ref, anch_ref, out_ref, sum_ref, side_ref):
    g = pl.program_id(0)
    A0 = xa_ref[...]
    A1 = xb_ref[...]
    A2 = xc_ref[...]
    T = tgt_ref[...]
    tci = jnp.floor(T[:, 0:1] * float(G)).astype(jnp.int32).reshape(B, 1, 1, 1)
    tcj = jnp.floor(T[:, 1:2] * float(G)).astype(jnp.int32).reshape(B, 1, 1, 1)
    g1 = lax.broadcasted_iota(jnp.int32, (1, 1, G, G), 2)
    g2 = lax.broadcasted_iota(jnp.int32, (1, 1, G, G), 3)
    cmask = jnp.where((g1 == tci) & (g2 == tcj), 1.0, 0.0)

    def colsum(A):
        return jnp.sum(jnp.sum(A * cmask, axis=3), axis=2).T

    side_ref[...] = jnp.concatenate(
        [colsum(A0)[..., None], colsum(A1)[..., None], colsum(A2)[..., None]],
        axis=2)

    def flat(P):
        return P.transpose(1, 0, 2, 3).reshape(5, B, GG)

    S0 = jax.nn.sigmoid(A0)
    S1 = jax.nn.sigmoid(A1)
    S2 = jax.nn.sigmoid(A2)

    @pl.when(g > 0)
    def _():
        out_ref[...] = jnp.concatenate([flat(S0), flat(S1), flat(S2)], axis=2)

    @pl.when(g == 0)
    def _():
        gxf = g2.astype(jnp.float32)
        gyf = g1.astype(jnp.float32)

        def head(S, a):
            r0 = (S[:, 0:1] + gxf) * STRIDE
            r1 = (S[:, 1:2] + gyf) * STRIDE
            r2 = jnp.exp(S[:, 2:3]) * (anch_ref[a, 0] * STRIDE)
            r3 = jnp.exp(S[:, 3:4]) * (anch_ref[a, 1] * STRIDE)
            return flat(jnp.concatenate([r0, r1, r2, r3, S[:, 4:5]], axis=1))

        out_ref[...] = jnp.concatenate([head(S0, 0), head(S1, 1), head(S2, 2)],
                                       axis=2)

        def sptot(A):
            z = A[:, 4:5]
            sp = jnp.maximum(z, 0.0) + jnp.log1p(jnp.exp(-jnp.abs(z)))
            return jnp.sum(jnp.minimum(sp, 100.0))

        sum_ref[0, 0] = sptot(A0) + sptot(A1) + sptot(A2)


def _tc_transform(x, anchors, target, interpret=False):
    xv = x.reshape(B, N_ANCHOR * CH, G, G)
    return pl.pallas_call(
        _tc_body,
        grid=(CH // 5,),
        in_specs=[
            pl.BlockSpec((B, 5, G, G), lambda g: (0, g, 0, 0)),
            pl.BlockSpec((B, 5, G, G), lambda g: (0, CH // 5 + g, 0, 0)),
            pl.BlockSpec((B, 5, G, G), lambda g: (0, 2 * (CH // 5) + g, 0, 0)),
            pl.BlockSpec(memory_space=pltpu.VMEM),
            pl.BlockSpec(memory_space=pltpu.SMEM),
        ],
        out_specs=[
            pl.BlockSpec((5, B, N_ANCHOR * GG), lambda g: (g, 0, 0)),
            pl.BlockSpec(memory_space=pltpu.SMEM),
            pl.BlockSpec((5, B, N_ANCHOR), lambda g: (g, 0, 0)),
        ],
        out_shape=[
            jax.ShapeDtypeStruct((CH, B, N_ANCHOR * GG), jnp.float32),
            jax.ShapeDtypeStruct((1, 1), jnp.float32),
            jax.ShapeDtypeStruct((CH, B, N_ANCHOR), jnp.float32),
        ],
        interpret=interpret,
    )(xv, xv, xv, target, anchors)


def _poly_log1p(w):
    s = w / (2.0 + w)
    s2 = s * s
    return 2.0 * s * (1.0 + s2 * (1.0 / 3.0 + s2 * (0.2 + s2 * (1.0 / 7.0 + s2 / 9.0))))


def _poly_log(x):
    bits = lax.bitcast_convert_type(x, jnp.int32)
    e = ((bits >> 23) & 0xFF) - 127
    mbits = (bits & 0x7FFFFF) | (127 << 23)
    m = lax.bitcast_convert_type(mbits, jnp.float32)
    big = m > 1.4142135
    m = jnp.where(big, m * 0.5, m)
    e = e + jnp.where(big, 1, 0)
    s = (m - 1.0) / (m + 1.0)
    s2 = s * s
    lm = 2.0 * s * (1.0 + s2 * (1.0 / 3.0 + s2 * (0.2 + s2 * (1.0 / 7.0 + s2 / 9.0))))
    return e.astype(jnp.float32) * 0.6931471805599453 + lm


def _softplus_nc(t):
    w = jnp.exp(-jnp.abs(t))
    return jnp.maximum(t, 0.0) + _poly_log1p(w)


def _softplus_c(t):
    return jnp.minimum(_softplus_nc(t), 100.0)


def _sigmoid_c(z):
    return 1.0 / (1.0 + jnp.exp(-z))


_NROW = CH + N_ANCHOR
_NIDX = _NROW * B


def _vsum(v):
    iot = lax.iota(jnp.int32, 16)
    for sh in (8, 4, 2, 1):
        perm = (iot + sh) & 15
        v = v + lax.gather(
            v, perm[:, None],
            lax.GatherDimensionNumbers(offset_dims=(), collapsed_slice_dims=(0,),
                                       start_index_map=(0,)),
            (1,), mode=lax.GatherScatterMode.PROMISE_IN_BOUNDS)
    return v


def _sc_body(x1, tgt, anc, out, tv, av, idxr, buf, resv, sem):
    wid = lax.axis_index("s") * 2 + lax.axis_index("c")

    @pl.when(wid == 0)
    def _():
        pltpu.sync_copy(tgt, tv)
        pltpu.sync_copy(anc, av)
        iot = lax.iota(jnp.int32, 16)
        f32 = jnp.float32

        tcx = tv[0, :] * float(G)
        tcy = tv[1, :] * float(G)
        tw = tv[2, :]
        th = tv[3, :]
        tci = tcx.astype(jnp.int32)
        tcj = tcy.astype(jnp.int32)
        fx = tcx - tci.astype(f32)
        fy = tcy - tcj.astype(f32)
        colc = tci * G + tcj
        ious = []
        for a in range(N_ANCHOR):
            aw = av[a, :]
            ah = av[N_ANCHOR + a, :]
            inter = jnp.minimum(aw, tw) * jnp.minimum(ah, th)
            union = tw * th + aw * ah - inter
            ious.append(inter / union)
        best = jnp.where(ious[1] > ious[0], 1, 0)
        best = jnp.where(ious[2] > jnp.maximum(ious[0], ious[1]), 2, best)
        aw_b = jnp.where(best == 0, av[0, :], jnp.where(best == 1, av[1, :], av[2, :]))
        ah_b = jnp.where(best == 0, av[3, :], jnp.where(best == 1, av[4, :], av[5, :]))

        fbase = iot * N_ANCHOR + best
        for j in range(CH):
            idxr[pl.ds(j * 16, 16)] = fbase + j * (B * N_ANCHOR)
        for a in range(N_ANCHOR):
            idxr[pl.ds((CH + a) * 16, 16)] = 4 * (B * N_ANCHOR) + iot * N_ANCHOR + a
        copies = []
        for ci in range(_NIDX // 128):
            copies.append(pltpu.async_copy(
                x1.at[idxr.at[pl.ds(ci * 128, 128)]],
                buf.at[pl.ds(ci * 128, 128)], sem))
        for cp in copies:
            cp.wait()

        def val(j):
            return buf[pl.ds(j * 16, 16)]

        d0 = _sigmoid_c(val(0)) - fx
        d1 = _sigmoid_c(val(1)) - fy
        d2 = _sigmoid_c(val(2)) - _poly_log(tw / aw_b + 1e-16)
        d3 = _sigmoid_c(val(3)) - _poly_log(th / ah_b + 1e-16)
        box_sum = _vsum(d0 * d0 + d1 * d1 + d2 * d2 + d3 * d3)
        objconf_sum = _vsum(_softplus_c(-val(4)))

        cacc = jnp.zeros((16,), f32)
        for j in range(5, CH):
            v = val(j)
            t = tv[j - 1, :]
            a_sp = _softplus_nc(-v)
            cacc = cacc + t * jnp.minimum(a_sp, 100.0) + (1.0 - t) * jnp.minimum(v + a_sp, 100.0)
        class_sum = _vsum(cacc)

        corr = jnp.zeros((16,), f32)
        cnt = jnp.zeros((16,), f32)
        for a in range(N_ANCHOR):
            z = val(CH + a)
            rem = (ious[a] > THRESH) | (best == a)
            corr = corr + jnp.where(rem, _softplus_c(z), 0.0)
            cnt = cnt + jnp.where(rem, 1.0, 0.0)
        corr_sum = _vsum(corr)
        removed = _vsum(cnt)

        sel = lambda k: jnp.where(iot == k, 1.0, 0.0)
        resv[...] = (box_sum * sel(0) + objconf_sum * sel(1) + class_sum * sel(2)
                     + corr_sum * sel(3) + removed * sel(4))
        pltpu.sync_copy(resv, out)


def _make_sc_loss():
    return functools.partial(
        pl.kernel,
        out_type=jax.ShapeDtypeStruct((16,), jnp.float32),
        mesh=plsc.VectorSubcoreMesh(core_axis_name="c", subcore_axis_name="s"),
        scratch_types=[
            pltpu.VMEM((84, B), jnp.float32),
            pltpu.VMEM((2 * N_ANCHOR, B), jnp.float32),
            pltpu.VMEM((_NIDX,), jnp.int32),
            pltpu.VMEM((_NIDX,), jnp.float32),
            pltpu.VMEM((16,), jnp.float32),
            pltpu.SemaphoreType.DMA,
        ],
    )(_sc_body)


def kernel(x, target, anchors):
    out_r, s_total, side = _tc_transform(x, anchors, target)
    output = jnp.transpose(out_r, (1, 2, 0))
    parts = _make_sc_loss()(
        side.reshape(CH * B * N_ANCHOR), target.T,
        jnp.broadcast_to(anchors.T.reshape(2 * N_ANCHOR, 1), (2 * N_ANCHOR, B)))
    box_sum, objconf_sum, class_sum, corr_sum, removed = (
        parts[0], parts[1], parts[2], parts[3], parts[4])
    loss = (box_sum / B + objconf_sum / B
            + NO_OBJ_W * (s_total[0, 0] - corr_sum) / (NCELL - removed)
            + class_sum / (B * N_CLASS))
    return output, loss
